# Initial kernel scaffold; baseline (speedup 1.0000x reference)
import jax
import jax.numpy as jnp
from jax.experimental import pallas as pl


def kernel(z, edge_index, p):
    raise NotImplementedError("write your pallas kernel here")


# speedup vs baseline: 1.3780x; 1.3780x over previous
import functools

import jax
import jax.numpy as jnp
from jax import lax
from jax.experimental import pallas as pl
from jax.experimental.pallas import tpu as pltpu
from jax.experimental.pallas import tpu_sc as plsc

E = 320000
N = 10000
D = 128
NC = 2
NS = 16
NW = NC * NS
EPW = E // NW
C = 400
NCHUNK = EPW // C
L = 16

CLIP = 1.0 + 1e-7


def _mdot_body(z_hbm, src_hbm, dst_hbm, out_hbm, sidx, didx, srows, drows,
               macc, sem):
    wid = lax.axis_index("s") * NC + lax.axis_index("c")
    base = wid * EPW
    pltpu.sync_copy(src_hbm.at[pl.ds(base, EPW)], sidx)
    pltpu.sync_copy(dst_hbm.at[pl.ds(base, EPW)], didx)

    def chunk_body(ci, _):
        off = ci * C
        cp_s = pltpu.async_copy(z_hbm.at[sidx.at[pl.ds(off, C)]], srows, sem)
        cp_d = pltpu.async_copy(z_hbm.at[didx.at[pl.ds(off, C)]], drows, sem)
        cp_s.wait()
        cp_d.wait()

        def group_body(g, _):
            lanes = g * L + lax.iota(jnp.int32, L)
            acc = jnp.zeros((L,), jnp.float32)
            for k in range(D):
                kv = jnp.full((L,), k, jnp.int32)
                s = plsc.load_gather(srows, [lanes, kv])
                d = plsc.load_gather(drows, [lanes, kv])
                if k == 0:
                    acc = acc - s * d
                else:
                    acc = acc + s * d
            macc[pl.ds(g * L, L)] = acc
            return 0

        lax.fori_loop(0, C // L, group_body, 0)
        pltpu.sync_copy(macc, out_hbm.at[pl.ds(base + off, C)])
        return 0

    lax.fori_loop(0, NCHUNK, chunk_body, 0)


@functools.cache
def _mdot_sc():
    return functools.partial(
        pl.kernel,
        out_type=jax.ShapeDtypeStruct((E,), jnp.float32),
        mesh=plsc.VectorSubcoreMesh(
            core_axis_name="c", subcore_axis_name="s",
            num_cores=NC, num_subcores=NS,
        ),
        scratch_types=[
            pltpu.VMEM((EPW,), jnp.int32),
            pltpu.VMEM((EPW,), jnp.int32),
            pltpu.VMEM((C, D), jnp.float32),
            pltpu.VMEM((C, D), jnp.float32),
            pltpu.VMEM((C,), jnp.float32),
            pltpu.SemaphoreType.DMA,
        ],
        compiler_params=pltpu.CompilerParams(needs_layout_passes=False),
    )(_mdot_body)


def _tail_body(mdot_ref, out_ref):
    md = mdot_ref[...]
    arg = jnp.maximum(-md, CLIP)
    dist = jnp.log(arg + jnp.sqrt((arg + 1.0) * (arg - 1.0)))
    d = 1.0 / dist
    m = jnp.max(d)
    out_ref[...] = jnp.exp(d - m)


def kernel(z, edge_index, p):
    del p
    mdot = _mdot_sc()(z, edge_index[0], edge_index[1])
    tail = pl.pallas_call(
        _tail_body,
        out_shape=jax.ShapeDtypeStruct((E // D, D), jnp.float32),
    )
    return tail(mdot.reshape(E // D, D)).reshape(E)


# speedup vs baseline: 1.3912x; 1.0096x over previous
import functools

import jax
import jax.numpy as jnp
from jax import lax
from jax.experimental import pallas as pl
from jax.experimental.pallas import tpu as pltpu
from jax.experimental.pallas import tpu_sc as plsc

E = 320000
N = 10000
D = 128
NC = 2
NS = 16
NW = NC * NS
EPW = E // NW
C = 80
NCHUNK = EPW // C
L = 16

CLIP = 1.0 + 1e-7


def _mdot_body(z_hbm, src_hbm, dst_hbm, out_hbm, zsh, sidx, didx, srows,
               drows, macc, sem):
    sid = lax.axis_index("s")
    wid = sid * NC + lax.axis_index("c")
    base = wid * EPW

    @pl.when(sid == 0)
    def _():
        pltpu.sync_copy(z_hbm, zsh)

    pltpu.sync_copy(src_hbm.at[pl.ds(base, EPW)], sidx)
    pltpu.sync_copy(dst_hbm.at[pl.ds(base, EPW)], didx)
    plsc.subcore_barrier()

    def chunk_body(ci, _):
        off = ci * C
        cp_s = pltpu.async_copy(zsh.at[sidx.at[pl.ds(off, C)]], srows, sem)
        cp_d = pltpu.async_copy(zsh.at[didx.at[pl.ds(off, C)]], drows, sem)
        cp_s.wait()
        cp_d.wait()

        def group_body(g, _):
            lanes = g * L + lax.iota(jnp.int32, L)
            acc = jnp.zeros((L,), jnp.float32)
            for k in range(D):
                kv = jnp.full((L,), k, jnp.int32)
                s = plsc.load_gather(srows, [lanes, kv])
                d = plsc.load_gather(drows, [lanes, kv])
                if k == 0:
                    acc = acc - s * d
                else:
                    acc = acc + s * d
            macc[pl.ds(g * L, L)] = acc
            return 0

        lax.fori_loop(0, C // L, group_body, 0)
        pltpu.sync_copy(macc, out_hbm.at[pl.ds(base + off, C)])
        return 0

    lax.fori_loop(0, NCHUNK, chunk_body, 0)


@functools.cache
def _mdot_sc():
    return functools.partial(
        pl.kernel,
        out_type=jax.ShapeDtypeStruct((E,), jnp.float32),
        mesh=plsc.VectorSubcoreMesh(
            core_axis_name="c", subcore_axis_name="s",
            num_cores=NC, num_subcores=NS,
        ),
        scratch_types=[
            pltpu.VMEM_SHARED((N, D), jnp.float32),
            pltpu.VMEM((EPW,), jnp.int32),
            pltpu.VMEM((EPW,), jnp.int32),
            pltpu.VMEM((C, D), jnp.float32),
            pltpu.VMEM((C, D), jnp.float32),
            pltpu.VMEM((C,), jnp.float32),
            pltpu.SemaphoreType.DMA,
        ],
        compiler_params=pltpu.CompilerParams(needs_layout_passes=False),
    )(_mdot_body)


def _tail_body(mdot_ref, out_ref):
    md = mdot_ref[...]
    arg = jnp.maximum(-md, CLIP)
    dist = jnp.log(arg + jnp.sqrt((arg + 1.0) * (arg - 1.0)))
    d = 1.0 / dist
    m = jnp.max(d)
    out_ref[...] = jnp.exp(d - m)


def kernel(z, edge_index, p):
    del p
    mdot = _mdot_sc()(z, edge_index[0], edge_index[1])
    tail = pl.pallas_call(
        _tail_body,
        out_shape=jax.ShapeDtypeStruct((E // D, D), jnp.float32),
    )
    return tail(mdot.reshape(E // D, D)).reshape(E)


# speedup vs baseline: 5.8113x; 4.1773x over previous
import functools

import jax
import jax.numpy as jnp
from jax import lax
from jax.experimental import pallas as pl
from jax.experimental.pallas import tpu as pltpu
from jax.experimental.pallas import tpu_sc as plsc

E = 320000
N = 10000
D = 128
NC = 2
NS = 16
NW = NC * NS
EPW = E // NW
C = 80
NCHUNK = EPW // C
L = 16

CLIP = 1.0 + 1e-7


def _mdot_body(z_hbm, src_hbm, dst_hbm, out_hbm, zsh, sidx, didx, srows,
               drows, macc, tbuf, sem):
    sid = lax.axis_index("s")
    wid = sid * NC + lax.axis_index("c")
    base = wid * EPW

    @pl.when(sid == 0)
    def _():
        pltpu.sync_copy(z_hbm, zsh)

    pltpu.sync_copy(src_hbm.at[pl.ds(base, EPW)], sidx)
    pltpu.sync_copy(dst_hbm.at[pl.ds(base, EPW)], didx)
    plsc.subcore_barrier()

    def chunk_body(ci, _):
        off = ci * C
        cp_s = pltpu.async_copy(zsh.at[sidx.at[pl.ds(off, C)]], srows, sem)
        cp_d = pltpu.async_copy(zsh.at[didx.at[pl.ds(off, C)]], drows, sem)
        cp_s.wait()
        cp_d.wait()

        def group_body(g, _):
            gbase = g * L
            lane = lax.iota(jnp.int32, L)
            sgn = jnp.where(lane == 0, -1.0, 1.0).astype(jnp.float32)

            for j in range(L):
                e = gbase + j
                prods = []
                for q in range(D // L):
                    s = srows[e, pl.ds(q * L, L)]
                    d = drows[e, pl.ds(q * L, L)]
                    prods.append(s * d * sgn if q == 0 else s * d)
                while len(prods) > 1:
                    prods = [a + b for a, b in zip(prods[::2], prods[1::2])]
                tbuf[pl.ds(j * (L + 1), L)] = prods[0]

            bidx = lane * (L + 1)
            cols = [plsc.load_gather(tbuf, [bidx + q]) for q in range(L)]
            while len(cols) > 1:
                cols = [a + b for a, b in zip(cols[::2], cols[1::2])]
            macc[pl.ds(gbase, L)] = cols[0]
            return 0

        lax.fori_loop(0, C // L, group_body, 0)
        pltpu.sync_copy(macc, out_hbm.at[pl.ds(base + off, C)])
        return 0

    lax.fori_loop(0, NCHUNK, chunk_body, 0)


@functools.cache
def _mdot_sc():
    return functools.partial(
        pl.kernel,
        out_type=jax.ShapeDtypeStruct((E,), jnp.float32),
        mesh=plsc.VectorSubcoreMesh(
            core_axis_name="c", subcore_axis_name="s",
            num_cores=NC, num_subcores=NS,
        ),
        scratch_types=[
            pltpu.VMEM_SHARED((N, D), jnp.float32),
            pltpu.VMEM((EPW,), jnp.int32),
            pltpu.VMEM((EPW,), jnp.int32),
            pltpu.VMEM((C, D), jnp.float32),
            pltpu.VMEM((C, D), jnp.float32),
            pltpu.VMEM((C,), jnp.float32),
            pltpu.VMEM((L * (L + 1),), jnp.float32),
            pltpu.SemaphoreType.DMA,
        ],
        compiler_params=pltpu.CompilerParams(needs_layout_passes=False),
    )(_mdot_body)


def _tail_body(mdot_ref, out_ref):
    md = mdot_ref[...]
    arg = jnp.maximum(-md, CLIP)
    dist = jnp.log(arg + jnp.sqrt((arg + 1.0) * (arg - 1.0)))
    d = 1.0 / dist
    m = jnp.max(d)
    out_ref[...] = jnp.exp(d - m)


def kernel(z, edge_index, p):
    del p
    mdot = _mdot_sc()(z, edge_index[0], edge_index[1])
    tail = pl.pallas_call(
        _tail_body,
        out_shape=jax.ShapeDtypeStruct((E // D, D), jnp.float32),
    )
    return tail(mdot.reshape(E // D, D)).reshape(E)


# speedup vs baseline: 8.6862x; 1.4947x over previous
import functools

import jax
import jax.numpy as jnp
from jax import lax
from jax.experimental import pallas as pl
from jax.experimental.pallas import tpu as pltpu
from jax.experimental.pallas import tpu_sc as plsc

E = 320000
N = 10000
D = 128
NC = 2
NS = 16
NW = NC * NS
EPW = E // NW
C = 80
NCHUNK = EPW // C
L = 16

CLIP = 1.0 + 1e-7


def _mdot_body(z_hbm, src_hbm, dst_hbm, out_hbm, zsh, cidx, srowsb, drowsb,
               maccb, tbuf, sem_rows, sem_idx, sem_out):
    sid = lax.axis_index("s")
    wid = sid * NC + lax.axis_index("c")
    base = wid * EPW

    @pl.when(sid == 0)
    def _():
        pltpu.sync_copy(z_hbm, zsh)

    pltpu.sync_copy(src_hbm.at[pl.ds(base, C)], cidx.at[0, 0])
    pltpu.sync_copy(dst_hbm.at[pl.ds(base, C)], cidx.at[0, 1])
    plsc.subcore_barrier()
    pltpu.async_copy(zsh.at[cidx.at[0, 0]], srowsb.at[0], sem_rows)
    pltpu.async_copy(zsh.at[cidx.at[0, 1]], drowsb.at[0], sem_rows)
    pltpu.async_copy(src_hbm.at[pl.ds(base + C, C)], cidx.at[1, 0], sem_idx)
    pltpu.async_copy(dst_hbm.at[pl.ds(base + C, C)], cidx.at[1, 1], sem_idx)

    def chunk_body(ci, _):
        b = lax.rem(ci, 2)
        nb = 1 - b
        pltpu.make_async_copy(zsh.at[cidx.at[b, 0]], srowsb.at[b],
                              sem_rows).wait()
        pltpu.make_async_copy(zsh.at[cidx.at[b, 1]], drowsb.at[b],
                              sem_rows).wait()

        @pl.when(ci + 1 < NCHUNK)
        def _():
            pltpu.make_async_copy(src_hbm.at[pl.ds(base, C)], cidx.at[nb, 0],
                                  sem_idx).wait()
            pltpu.make_async_copy(dst_hbm.at[pl.ds(base, C)], cidx.at[nb, 1],
                                  sem_idx).wait()
            pltpu.async_copy(zsh.at[cidx.at[nb, 0]], srowsb.at[nb], sem_rows)
            pltpu.async_copy(zsh.at[cidx.at[nb, 1]], drowsb.at[nb], sem_rows)

        @pl.when(ci + 2 < NCHUNK)
        def _():
            off2 = base + (ci + 2) * C
            pltpu.async_copy(src_hbm.at[pl.ds(off2, C)], cidx.at[b, 0],
                             sem_idx)
            pltpu.async_copy(dst_hbm.at[pl.ds(off2, C)], cidx.at[b, 1],
                             sem_idx)

        @pl.when(ci >= 2)
        def _():
            pltpu.make_async_copy(maccb.at[b], out_hbm.at[pl.ds(base, C)],
                                  sem_out).wait()

        def group_body(g, _):
            gbase = g * L
            lane = lax.iota(jnp.int32, L)
            sgn = jnp.where(lane == 0, -1.0, 1.0).astype(jnp.float32)

            for j in range(L):
                e = gbase + j
                prods = []
                for q in range(D // L):
                    s = srowsb[b, e, pl.ds(q * L, L)]
                    d = drowsb[b, e, pl.ds(q * L, L)]
                    prods.append(s * d * sgn if q == 0 else s * d)
                while len(prods) > 1:
                    prods = [a + b2 for a, b2 in zip(prods[::2], prods[1::2])]
                tbuf[pl.ds(j * (L + 1), L)] = prods[0]

            bidx = lane * (L + 1)
            cols = [plsc.load_gather(tbuf, [bidx + q]) for q in range(L)]
            while len(cols) > 1:
                cols = [a + b2 for a, b2 in zip(cols[::2], cols[1::2])]
            maccb[b, pl.ds(gbase, L)] = cols[0]
            return 0

        lax.fori_loop(0, C // L, group_body, 0)
        pltpu.async_copy(maccb.at[b], out_hbm.at[pl.ds(base + ci * C, C)],
                         sem_out)
        return 0

    lax.fori_loop(0, NCHUNK, chunk_body, 0)

    pltpu.make_async_copy(maccb.at[0], out_hbm.at[pl.ds(base, C)],
                          sem_out).wait()
    pltpu.make_async_copy(maccb.at[1], out_hbm.at[pl.ds(base, C)],
                          sem_out).wait()


@functools.cache
def _mdot_sc():
    return functools.partial(
        pl.kernel,
        out_type=jax.ShapeDtypeStruct((E,), jnp.float32),
        mesh=plsc.VectorSubcoreMesh(
            core_axis_name="c", subcore_axis_name="s",
            num_cores=NC, num_subcores=NS,
        ),
        scratch_types=[
            pltpu.VMEM_SHARED((N, D), jnp.float32),
            pltpu.VMEM((2, 2, C), jnp.int32),
            pltpu.VMEM((2, C, D), jnp.float32),
            pltpu.VMEM((2, C, D), jnp.float32),
            pltpu.VMEM((2, C), jnp.float32),
            pltpu.VMEM((L * (L + 1),), jnp.float32),
            pltpu.SemaphoreType.DMA,
            pltpu.SemaphoreType.DMA,
            pltpu.SemaphoreType.DMA,
        ],
        compiler_params=pltpu.CompilerParams(needs_layout_passes=False),
    )(_mdot_body)


def _tail_body(mdot_ref, out_ref):
    md = mdot_ref[...]
    arg = jnp.maximum(-md, CLIP)
    dist = jnp.log(arg + jnp.sqrt((arg + 1.0) * (arg - 1.0)))
    d = 1.0 / dist
    m = jnp.max(d)
    out_ref[...] = jnp.exp(d - m)


def kernel(z, edge_index, p):
    del p
    mdot = _mdot_sc()(z, edge_index[0], edge_index[1])
    tail = pl.pallas_call(
        _tail_body,
        out_shape=jax.ShapeDtypeStruct((E // D, D), jnp.float32),
    )
    return tail(mdot.reshape(E // D, D)).reshape(E)
